# Initial kernel scaffold; baseline (speedup 1.0000x reference)
import jax
import jax.numpy as jnp
from jax.experimental import pallas as pl


def kernel(melody, chord_ids, annotation_1, emb_melody, W_melody, b_melody, emb_chord, W_chord, b_chord, emb_ann, W_ann, b_ann):
    raise NotImplementedError("write your pallas kernel here")


# speedup vs baseline: 1.1899x; 1.1899x over previous
import functools

import jax
import jax.numpy as jnp
from jax import lax
from jax.experimental import pallas as pl
from jax.experimental.pallas import tpu as pltpu
from jax.experimental.pallas import tpu_sc as plsc

B, L, D = 2, 512, 64
_MAXP = (128, 395, 52)
_ROWS = (257, 791, 105)
_RPAD = (264, 792, 112)
_OFF = (0, 264, 1056)
_TAB = 1168


def _proj_body(em, wm, bm, ec, wc, bc, ea, wa, ba, om, oc, oa):
    dn = (((1,), (1,)), ((), ()))
    om[...] = lax.dot_general(em[...], wm[...], dn,
                              preferred_element_type=jnp.float32) + bm[...]
    oc[...] = lax.dot_general(ec[...], wc[...], dn,
                              preferred_element_type=jnp.float32) + bc[...]
    oa[...] = lax.dot_general(ea[...], wa[...], dn,
                              preferred_element_type=jnp.float32) + ba[...]


def _make_sc_kernel():
    info = plsc.get_sparse_core_info()
    nc, ns = info.num_cores, info.num_subcores
    nw = nc * ns
    rows_per_w = (B * L) // nw
    mesh = plsc.VectorSubcoreMesh(core_axis_name="c", subcore_axis_name="s")

    out_type = [jax.ShapeDtypeStruct((B * L * L * D,), jnp.float32)
                for _ in range(3)]
    scratch = [
        pltpu.VMEM((_TAB * D,), jnp.float32),
        pltpu.VMEM((3 * B * L,), jnp.int32),
        pltpu.VMEM((L * D,), jnp.float32),
    ]

    @functools.partial(
        pl.kernel, mesh=mesh, out_type=out_type, scratch_types=scratch,
        compiler_params=pltpu.CompilerParams(needs_layout_passes=False))
    def sc(pm, pc, pa, im, ic, ia, om, oc, oa, tab_v, ids_v, stage_v):
        wid = lax.axis_index("s") * nc + lax.axis_index("c")
        pltpu.sync_copy(pm, tab_v.at[pl.ds(_OFF[0] * D, _RPAD[0] * D)])
        pltpu.sync_copy(pc, tab_v.at[pl.ds(_OFF[1] * D, _RPAD[1] * D)])
        pltpu.sync_copy(pa, tab_v.at[pl.ds(_OFF[2] * D, _RPAD[2] * D)])
        pltpu.sync_copy(im, ids_v.at[pl.ds(0, B * L)])
        pltpu.sync_copy(ic, ids_v.at[pl.ds(B * L, B * L)])
        pltpu.sync_copy(ia, ids_v.at[pl.ds(2 * B * L, B * L)])

        lane = lax.iota(jnp.int32, 16)
        outs = (om, oc, oa)
        for s in range(3):
            m = _MAXP[s]
            shift = m + _OFF[s]
            out_ref = outs[s]

            def row_body(r, _, s=s, m=m, shift=shift, out_ref=out_ref):
                rr = wid + nw * r
                b = rr // L
                i = rr % L
                ib = s * (B * L) + b * L
                idv = plsc.load_gather(ids_v, [jnp.full((16,), ib + i,
                                                        jnp.int32)])
                for c in range(L // 16):
                    v = ids_v[pl.ds(ib + c * 16, 16)]
                    posd = (jnp.clip(idv - v, -m, m) + shift) * D
                    jbase = (lane + c * 16) * D

                    def d_body(dd, _, posd=posd, jbase=jbase):
                        col = jnp.full((16,), dd, jnp.int32)
                        vals = plsc.load_gather(tab_v, [posd + col])
                        plsc.store_scatter(stage_v, [jbase + col], vals)
                        return 0

                    lax.fori_loop(0, D, d_body, 0, unroll=4)
                pltpu.sync_copy(stage_v, out_ref.at[pl.ds(rr * (L * D),
                                                          L * D)])
                return 0

            lax.fori_loop(0, rows_per_w, row_body, 0)

    return sc


_sc_kernel = _make_sc_kernel()


def kernel(melody, chord_ids, annotation_1,
           emb_melody, W_melody, b_melody,
           emb_chord, W_chord, b_chord,
           emb_ann, W_ann, b_ann):
    em = jnp.pad(emb_melody, ((0, _RPAD[0] - _ROWS[0]), (0, 0)))
    ec = jnp.pad(emb_chord, ((0, _RPAD[1] - _ROWS[1]), (0, 0)))
    ea = jnp.pad(emb_ann, ((0, _RPAD[2] - _ROWS[2]), (0, 0)))
    pm, pc, pa = pl.pallas_call(
        _proj_body,
        out_shape=[jax.ShapeDtypeStruct((_RPAD[0], D), jnp.float32),
                   jax.ShapeDtypeStruct((_RPAD[1], D), jnp.float32),
                   jax.ShapeDtypeStruct((_RPAD[2], D), jnp.float32)],
    )(em, W_melody, b_melody.reshape(1, D),
      ec, W_chord, b_chord.reshape(1, D),
      ea, W_ann, b_ann.reshape(1, D))

    im = melody.reshape(B * L).astype(jnp.int32)
    ic = chord_ids.reshape(B * L).astype(jnp.int32)
    ia = annotation_1.reshape(B * L).astype(jnp.int32)
    om, oc, oa = _sc_kernel(pm.reshape(-1), pc.reshape(-1), pa.reshape(-1),
                            im, ic, ia)
    shp = (B, L, L, D)
    return (om.reshape(shp), oc.reshape(shp), oa.reshape(shp))


# speedup vs baseline: 2.6677x; 2.2419x over previous
import functools

import jax
import jax.numpy as jnp
from jax import lax
from jax.experimental import pallas as pl
from jax.experimental.pallas import tpu as pltpu
from jax.experimental.pallas import tpu_sc as plsc

B, L, D = 2, 512, 64
_MAXP = (128, 395, 52)
_ROWS = (257, 791, 105)
_RPAD = (264, 792, 112)
_CH = 128


def _proj_body(em, wm, bm, ec, wc, bc, ea, wa, ba, om, oc, oa):
    dn = (((1,), (1,)), ((), ()))
    om[...] = lax.dot_general(em[...], wm[...], dn,
                              preferred_element_type=jnp.float32) + bm[...]
    oc[...] = lax.dot_general(ec[...], wc[...], dn,
                              preferred_element_type=jnp.float32) + bc[...]
    oa[...] = lax.dot_general(ea[...], wa[...], dn,
                              preferred_element_type=jnp.float32) + ba[...]


def _make_sc_kernel():
    info = plsc.get_sparse_core_info()
    nc, ns = info.num_cores, info.num_subcores
    nw = nc * ns
    rows_per_phase = (B * L) // nw
    mesh = plsc.VectorSubcoreMesh(core_axis_name="c", subcore_axis_name="s")

    out_type = [jax.ShapeDtypeStruct((B * L * L, D), jnp.float32)
                for _ in range(3)]
    scratch = [
        pltpu.VMEM((3 * B * L,), jnp.int32),
        pltpu.VMEM((2, L // _CH, _CH), jnp.int32),
        pltpu.VMEM((2, L, D), jnp.float32),
        pltpu.SemaphoreType.DMA,
        pltpu.SemaphoreType.DMA,
        pltpu.SemaphoreType.DMA,
    ]

    @functools.partial(
        pl.kernel, mesh=mesh, out_type=out_type, scratch_types=scratch,
        compiler_params=pltpu.CompilerParams(needs_layout_passes=False,
                                             use_tc_tiling_on_sc=False))
    def sc(pm, pc, pa, im, ic, ia, om, oc, oa,
           ids_v, idx_v, stage_v, gsem, osem0, osem1):
        wid = lax.axis_index("s") * nc + lax.axis_index("c")
        pltpu.sync_copy(im, ids_v.at[pl.ds(0, B * L)])
        pltpu.sync_copy(ic, ids_v.at[pl.ds(B * L, B * L)])
        pltpu.sync_copy(ia, ids_v.at[pl.ds(2 * B * L, B * L)])

        tabs = (pm, pc, pa)
        outs = (om, oc, oa)
        osems = (osem0, osem1)
        for s in range(3):
            m = _MAXP[s]
            tab_ref = tabs[s]
            out_ref = outs[s]

            def pair_body(p, _, s=s, m=m, tab_ref=tab_ref, out_ref=out_ref):
                for x in (0, 1):
                    k = 2 * p + x
                    rr = wid + nw * k
                    b = rr // L
                    i = rr % L
                    ib = s * (B * L) + b * L

                    @pl.when(p > 0)
                    def _():
                        pltpu.make_async_copy(
                            stage_v.at[x], out_ref.at[pl.ds(0, L)],
                            osems[x]).wait()

                    idv = plsc.load_gather(
                        ids_v, [jnp.full((16,), ib + i, jnp.int32)])
                    for cc in range(L // 16):
                        v = ids_v[pl.ds(ib + cc * 16, 16)]
                        pos = jnp.clip(idv - v, -m, m) + m
                        c, t = divmod(cc, _CH // 16)
                        idx_v[x, c, pl.ds(t * 16, 16)] = pos

                    copies = [
                        pltpu.async_copy(
                            tab_ref.at[idx_v.at[x, c]],
                            stage_v.at[x, pl.ds(c * _CH, _CH)], gsem)
                        for c in range(L // _CH)
                    ]
                    for cp in copies:
                        cp.wait()

                    pltpu.async_copy(
                        stage_v.at[x], out_ref.at[pl.ds(rr * L, L)], osems[x])
                return 0

            lax.fori_loop(0, rows_per_phase // 2, pair_body, 0)

            for x in (0, 1):
                pltpu.make_async_copy(
                    stage_v.at[x], out_ref.at[pl.ds(0, L)], osems[x]).wait()

    return sc


_sc_kernel = _make_sc_kernel()


def kernel(melody, chord_ids, annotation_1,
           emb_melody, W_melody, b_melody,
           emb_chord, W_chord, b_chord,
           emb_ann, W_ann, b_ann):
    em = jnp.pad(emb_melody, ((0, _RPAD[0] - _ROWS[0]), (0, 0)))
    ec = jnp.pad(emb_chord, ((0, _RPAD[1] - _ROWS[1]), (0, 0)))
    ea = jnp.pad(emb_ann, ((0, _RPAD[2] - _ROWS[2]), (0, 0)))
    pm, pc, pa = pl.pallas_call(
        _proj_body,
        out_shape=[jax.ShapeDtypeStruct((_RPAD[0], D), jnp.float32),
                   jax.ShapeDtypeStruct((_RPAD[1], D), jnp.float32),
                   jax.ShapeDtypeStruct((_RPAD[2], D), jnp.float32)],
    )(em, W_melody, b_melody.reshape(1, D),
      ec, W_chord, b_chord.reshape(1, D),
      ea, W_ann, b_ann.reshape(1, D))

    im = melody.reshape(B * L).astype(jnp.int32)
    ic = chord_ids.reshape(B * L).astype(jnp.int32)
    ia = annotation_1.reshape(B * L).astype(jnp.int32)
    om, oc, oa = _sc_kernel(pm, pc, pa, im, ic, ia)
    shp = (B, L, L, D)
    return (om.reshape(shp), oc.reshape(shp), oa.reshape(shp))


# speedup vs baseline: 5.3425x; 2.0027x over previous
import functools

import jax
import jax.numpy as jnp
from jax import lax
from jax.experimental import pallas as pl
from jax.experimental.pallas import tpu as pltpu
from jax.experimental.pallas import tpu_sc as plsc

B, L, D = 2, 512, 64
_MAXP = (128, 395, 52)
_ROWS = (257, 791, 105)
_RPAD = (264, 792, 112)
_CH = 128


def _proj_body(em, wm, bm, ec, wc, bc, ea, wa, ba, om, oc, oa):
    dn = (((1,), (1,)), ((), ()))
    om[...] = lax.dot_general(em[...], wm[...], dn,
                              preferred_element_type=jnp.float32) + bm[...]
    oc[...] = lax.dot_general(ec[...], wc[...], dn,
                              preferred_element_type=jnp.float32) + bc[...]
    oa[...] = lax.dot_general(ea[...], wa[...], dn,
                              preferred_element_type=jnp.float32) + ba[...]


def _make_sc_kernel():
    info = plsc.get_sparse_core_info()
    nc, ns = info.num_cores, info.num_subcores
    nw = nc * ns
    rows_per_phase = (B * L) // nw
    mesh = plsc.VectorSubcoreMesh(core_axis_name="c", subcore_axis_name="s")

    out_type = [jax.ShapeDtypeStruct((B * L * L, D), jnp.float32)
                for _ in range(3)]
    scratch = [
        pltpu.VMEM((3 * B * L,), jnp.int32),
        pltpu.VMEM((2, L // _CH, _CH), jnp.int32),
        pltpu.VMEM((2, L, D), jnp.float32),
        pltpu.VMEM_SHARED((sum(_RPAD), D), jnp.float32),
        pltpu.SemaphoreType.DMA,
        pltpu.SemaphoreType.DMA,
        pltpu.SemaphoreType.DMA,
    ]

    @functools.partial(
        pl.kernel, mesh=mesh, out_type=out_type, scratch_types=scratch,
        compiler_params=pltpu.CompilerParams(needs_layout_passes=False,
                                             use_tc_tiling_on_sc=False))
    def sc(pm, pc, pa, im, ic, ia, om, oc, oa,
           ids_v, idx_v, stage_v, tab_sh, gsem, osem0, osem1):
        wid = lax.axis_index("s") * nc + lax.axis_index("c")
        sid = lax.axis_index("s")
        pltpu.sync_copy(im, ids_v.at[pl.ds(0, B * L)])
        pltpu.sync_copy(ic, ids_v.at[pl.ds(B * L, B * L)])
        pltpu.sync_copy(ia, ids_v.at[pl.ds(2 * B * L, B * L)])

        @pl.when(sid == 0)
        def _():
            pltpu.sync_copy(pm, tab_sh.at[pl.ds(0, _RPAD[0])])
            pltpu.sync_copy(pc, tab_sh.at[pl.ds(_RPAD[0], _RPAD[1])])
            pltpu.sync_copy(pa, tab_sh.at[pl.ds(_RPAD[0] + _RPAD[1],
                                                _RPAD[2])])
        plsc.subcore_barrier()

        tabs = (pm, pc, pa)
        outs = (om, oc, oa)
        osems = (osem0, osem1)
        del tabs
        soffs = (0, _RPAD[0], _RPAD[0] + _RPAD[1])
        for s in range(3):
            m = _MAXP[s]
            soff = soffs[s]
            out_ref = outs[s]

            def pair_body(p, _, s=s, m=m, soff=soff, out_ref=out_ref):
                for x in (0, 1):
                    k = 2 * p + x
                    rr = wid + nw * k
                    b = rr // L
                    i = rr % L
                    ib = s * (B * L) + b * L

                    @pl.when(p > 0)
                    def _():
                        pltpu.make_async_copy(
                            stage_v.at[x], out_ref.at[pl.ds(0, L)],
                            osems[x]).wait()

                    idv = plsc.load_gather(
                        ids_v, [jnp.full((16,), ib + i, jnp.int32)])
                    for cc in range(L // 16):
                        v = ids_v[pl.ds(ib + cc * 16, 16)]
                        pos = jnp.clip(idv - v, -m, m) + (m + soff)
                        c, t = divmod(cc, _CH // 16)
                        idx_v[x, c, pl.ds(t * 16, 16)] = pos

                    copies = [
                        pltpu.async_copy(
                            tab_sh.at[idx_v.at[x, c]],
                            stage_v.at[x, pl.ds(c * _CH, _CH)], gsem)
                        for c in range(L // _CH)
                    ]
                    for cp in copies:
                        cp.wait()

                    pltpu.async_copy(
                        stage_v.at[x], out_ref.at[pl.ds(rr * L, L)], osems[x])
                return 0

            lax.fori_loop(0, rows_per_phase // 2, pair_body, 0)

            for x in (0, 1):
                pltpu.make_async_copy(
                    stage_v.at[x], out_ref.at[pl.ds(0, L)], osems[x]).wait()

    return sc


_sc_kernel = _make_sc_kernel()


def kernel(melody, chord_ids, annotation_1,
           emb_melody, W_melody, b_melody,
           emb_chord, W_chord, b_chord,
           emb_ann, W_ann, b_ann):
    em = jnp.pad(emb_melody, ((0, _RPAD[0] - _ROWS[0]), (0, 0)))
    ec = jnp.pad(emb_chord, ((0, _RPAD[1] - _ROWS[1]), (0, 0)))
    ea = jnp.pad(emb_ann, ((0, _RPAD[2] - _ROWS[2]), (0, 0)))
    pm, pc, pa = pl.pallas_call(
        _proj_body,
        out_shape=[jax.ShapeDtypeStruct((_RPAD[0], D), jnp.float32),
                   jax.ShapeDtypeStruct((_RPAD[1], D), jnp.float32),
                   jax.ShapeDtypeStruct((_RPAD[2], D), jnp.float32)],
    )(em, W_melody, b_melody.reshape(1, D),
      ec, W_chord, b_chord.reshape(1, D),
      ea, W_ann, b_ann.reshape(1, D))

    im = melody.reshape(B * L).astype(jnp.int32)
    ic = chord_ids.reshape(B * L).astype(jnp.int32)
    ia = annotation_1.reshape(B * L).astype(jnp.int32)
    om, oc, oa = _sc_kernel(pm, pc, pa, im, ic, ia)
    shp = (B, L, L, D)
    return (om.reshape(shp), oc.reshape(shp), oa.reshape(shp))
